# baseline (speedup 1.0000x reference)
import functools

import jax
import jax.numpy as jnp
from jax import lax
from jax.experimental import pallas as pl
from jax.experimental.pallas import tpu as pltpu
from jax.experimental.pallas import tpu_sc as plsc

_N = 10000
_E = 320000
_EPS = 1e-5

_NC = 2
_NS = 16
_CH = 80
_EPT = _E // (_NC * _NS)
_NCHUNK = _EPT // _CH
_RPT = 624
_RPT_LAST = _N - 15 * _RPT

_NBUF = 4


def _zero_acc(zeros_hbm, acc, s):
  @pl.when(s < _NS - 1)
  def _():
    pltpu.sync_copy(zeros_hbm.at[pl.ds(s * _RPT, _RPT)],
                    acc.at[pl.ds(s * _RPT, _RPT)])
  @pl.when(s == _NS - 1)
  def _():
    pltpu.sync_copy(zeros_hbm.at[pl.ds(15 * _RPT, _RPT_LAST)],
                    acc.at[pl.ds(15 * _RPT, _RPT_LAST)])


def _copy_out(acc, out_hbm, c, s):
  @pl.when(s < _NS - 1)
  def _():
    pltpu.sync_copy(acc.at[pl.ds(s * _RPT, _RPT)],
                    out_hbm.at[c, pl.ds(s * _RPT, _RPT)])
  @pl.when(s == _NS - 1)
  def _():
    pltpu.sync_copy(acc.at[pl.ds(15 * _RPT, _RPT_LAST)],
                    out_hbm.at[c, pl.ds(15 * _RPT, _RPT_LAST)])


@functools.lru_cache(maxsize=None)
def _make_sc_scatter(D):
  mesh = plsc.VectorSubcoreMesh(core_axis_name="c", subcore_axis_name="s",
                                num_cores=_NC, num_subcores=_NS)

  NB2 = 2 * _NBUF

  @functools.partial(
      pl.kernel,
      mesh=mesh,
      compiler_params=pltpu.CompilerParams(use_tc_tiling_on_sc=False),
      out_type=jax.ShapeDtypeStruct((_NC, _N, D), jnp.float32),
      scratch_types=[
          pltpu.VMEM((_NCHUNK, _CH), jnp.int32),
          pltpu.VMEM((_NCHUNK, _CH), jnp.int32),
          [pltpu.VMEM((_CH, D), jnp.float32) for _ in range(NB2)],
          pltpu.VMEM_SHARED((_N, D), jnp.float32),
          [pltpu.SemaphoreType.DMA for _ in range(NB2)],
          [pltpu.SemaphoreType.DMA for _ in range(NB2)],
      ],
  )
  def scat(xs_hbm, src_hbm, dst_hbm, zeros_hbm, out_hbm,
           sidx, didx, rows, acc, gsem, ssem):
    c = lax.axis_index("c")
    s = lax.axis_index("s")
    wid = c * _NS + s
    _zero_acc(zeros_hbm, acc, s)
    pltpu.sync_copy(src_hbm.at[wid], sidx)
    pltpu.sync_copy(dst_hbm.at[wid], didx)
    plsc.subcore_barrier()

    for b in range(_NBUF):
      pltpu.async_copy(xs_hbm.at[sidx.at[b]], rows[b], gsem[b])

    def group(g, carry):
      for b in range(NB2):
        j = g * NB2 + b
        qn = (b + _NBUF) % NB2
        @pl.when(j < _NCHUNK)
        def _():
          pltpu.make_async_copy(xs_hbm.at[sidx.at[j]], rows[b], gsem[b]).wait()
          pltpu.async_copy(rows[b], acc.at[didx.at[j]], ssem[b], add=True)
          @pl.when(j + _NBUF < _NCHUNK)
          def _():
            @pl.when(j >= _NBUF)
            def _():
              pltpu.make_async_copy(rows[qn], acc.at[didx.at[j]],
                                    ssem[qn]).wait()
            pltpu.async_copy(xs_hbm.at[sidx.at[j + _NBUF]], rows[qn], gsem[qn])
      return carry

    lax.fori_loop(0, (_NCHUNK + NB2 - 1) // NB2, group, 0, unroll=False)
    for k in range(_NCHUNK - NB2, _NCHUNK):
      q = k % NB2
      pltpu.make_async_copy(rows[q], acc.at[didx.at[0]], ssem[q]).wait()
    plsc.subcore_barrier()
    _copy_out(acc, out_hbm, c, s)

  return scat


@functools.lru_cache(maxsize=None)
def _make_sc_count(D):
  mesh = plsc.VectorSubcoreMesh(core_axis_name="c", subcore_axis_name="s",
                                num_cores=_NC, num_subcores=_NS)

  @functools.partial(
      pl.kernel,
      mesh=mesh,
      compiler_params=pltpu.CompilerParams(use_tc_tiling_on_sc=False),
      out_type=jax.ShapeDtypeStruct((_NC, _N, D), jnp.float32),
      scratch_types=[
          pltpu.VMEM((_NCHUNK, _CH), jnp.int32),
          pltpu.VMEM((_CH, D), jnp.float32),
          pltpu.VMEM_SHARED((_N, D), jnp.float32),
          [pltpu.SemaphoreType.DMA for _ in range(_NBUF)],
      ],
  )
  def count(ones_hbm, dst_hbm, zeros_hbm, out_hbm, didx, ones_v, acc, ssem):
    c = lax.axis_index("c")
    s = lax.axis_index("s")
    wid = c * _NS + s
    _zero_acc(zeros_hbm, acc, s)
    pltpu.sync_copy(dst_hbm.at[wid], didx)
    pltpu.sync_copy(ones_hbm, ones_v)
    plsc.subcore_barrier()

    def group(g, carry):
      for b in range(_NBUF):
        j = g * _NBUF + b
        @pl.when(j < _NCHUNK)
        def _():
          @pl.when(j >= _NBUF)
          def _():
            pltpu.make_async_copy(ones_v, acc.at[didx.at[j]], ssem[b]).wait()
          pltpu.async_copy(ones_v, acc.at[didx.at[j]], ssem[b], add=True)
      return carry

    lax.fori_loop(0, (_NCHUNK + _NBUF - 1) // _NBUF, group, 0, unroll=False)
    for b in range(_NBUF):
      j_last = _NCHUNK - _NBUF + b
      pltpu.make_async_copy(ones_v, acc.at[didx.at[j_last]], ssem[(j_last) % _NBUF]).wait()
    plsc.subcore_barrier()
    _copy_out(acc, out_hbm, c, s)

  return count


def _k1a_body(x_ref, w_ref, xw_ref):
  xw_ref[...] = jnp.dot(x_ref[...], w_ref[...],
                        preferred_element_type=jnp.float32)


def _k1b_body(xw_ref, d0_ref, d1_ref, xs_ref, dinv_ref):
  deg = d0_ref[...][:, 0:1] + d1_ref[...][:, 0:1] + 1.0
  dinv = lax.rsqrt(deg)
  xs_ref[...] = xw_ref[...] * dinv
  dinv_ref[...] = dinv


def _norm_affine(hc, bng, lng):
  m1 = jnp.mean(hc, axis=0, keepdims=True)
  v1 = jnp.maximum(jnp.mean(hc * hc, axis=0, keepdims=True) - m1 * m1, 0.0)
  a1 = bng * lax.rsqrt(v1 + _EPS)
  A = a1 * lax.rsqrt(a1 * a1 * v1 + _EPS)
  gv = jnp.mean(A * A * v1)
  F = A * lng * lax.rsqrt(gv + _EPS)
  return F, m1


def _klayer_body(p0_ref, p1_ref, xs_ref, dinv_ref, b_ref, bng_ref,
                 lng_ref, lnb_ref, w_ref, out_ref):
  dinv = dinv_ref[...]
  hc = (p0_ref[...] + p1_ref[...] + xs_ref[...]) * dinv + b_ref[...]
  F, m1 = _norm_affine(hc, bng_ref[...], lng_ref[...])
  h = jnp.maximum(F * (hc - m1) + lnb_ref[...], 0.0)
  xw = jnp.dot(h, w_ref[...], preferred_element_type=jnp.float32) * dinv
  if out_ref.shape[1] != xw.shape[1]:
    xw = jnp.broadcast_to(xw, out_ref.shape)
  out_ref[...] = xw


def _kfinal_body(p0_ref, p1_ref, xs_ref, dinv_ref, b_ref, bng_ref,
                 lng_ref, lnb_ref, out_ref):
  mp = p0_ref[...][:, 0:1] + p1_ref[...][:, 0:1] + xs_ref[...][:, 0:1]
  hc = mp * dinv_ref[...] + b_ref[...]
  F, m1 = _norm_affine(hc, bng_ref[...], lng_ref[...])
  out_ref[...] = F * (hc - m1) + lnb_ref[...]


def _tc_call(body, out_shape):
  return pl.pallas_call(body, out_shape=out_shape)


def kernel(x, edge_index, W1, b1, W2, b2, W3, b3, bn1_g, bn1_b, bn2_g,
           bn2_b, bn3_g, bn3_b, ln1_g, ln1_b, ln2_g, ln2_b, ln3_g, ln3_b):
  f32 = jnp.float32
  src = edge_index[0].reshape(_NC * _NS, _NCHUNK, _CH)
  dst = edge_index[1].reshape(_NC * _NS, _NCHUNK, _CH)
  zeros64 = jnp.zeros((_N, 64), f32)
  zeros8 = jnp.zeros((_N, 8), f32)
  ones8 = jnp.ones((_CH, 8), f32)

  sc64 = _make_sc_scatter(64)
  sc8 = _make_sc_scatter(8)

  degp = _make_sc_count(8)(ones8, dst, zeros8)
  xw1 = _tc_call(_k1a_body, jax.ShapeDtypeStruct((_N, 64), f32))(x, W1)

  xs1, dinv = _tc_call(
      _k1b_body,
      (jax.ShapeDtypeStruct((_N, 64), f32), jax.ShapeDtypeStruct((_N, 1), f32)),
  )(xw1, degp[0], degp[1])

  p1 = sc64(xs1, src, dst, zeros64)
  xs2 = _tc_call(_klayer_body, jax.ShapeDtypeStruct((_N, 64), f32))(
      p1[0], p1[1], xs1, dinv, b1.reshape(1, 64), bn1_g.reshape(1, 64),
      ln1_g.reshape(1, 64), ln1_b.reshape(1, 64), W2)

  p2 = sc64(xs2, src, dst, zeros64)
  xs3 = _tc_call(_klayer_body, jax.ShapeDtypeStruct((_N, 8), f32))(
      p2[0], p2[1], xs2, dinv, b2.reshape(1, 64), bn2_g.reshape(1, 64),
      ln2_g.reshape(1, 64), ln2_b.reshape(1, 64), W3)

  p3 = sc8(xs3, src, dst, zeros8)
  out = _tc_call(_kfinal_body, jax.ShapeDtypeStruct((_N, 1), f32))(
      p3[0], p3[1], xs3, dinv, b3.reshape(1, 1), bn3_g.reshape(1, 1),
      ln3_g.reshape(1, 1), ln3_b.reshape(1, 1))
  return out.reshape(-1)


# speedup vs baseline: 1.0464x; 1.0464x over previous
import functools

import jax
import jax.numpy as jnp
from jax import lax
from jax.experimental import pallas as pl
from jax.experimental.pallas import tpu as pltpu
from jax.experimental.pallas import tpu_sc as plsc

_N = 10000
_E = 320000
_EPS = 1e-5

_NC = 2
_NS = 16
_NT = _NC * _NS
_CH = 128
_ROWS = _E // _CH
_CPT = _ROWS // _NT
_EXTRA = _ROWS - _NT * _CPT
_MAXCH = _CPT + 1
_RPT = 624
_RPT_LAST = _N - 15 * _RPT

_NBUF = 4


def _zero_acc(zeros_hbm, acc, s):
  @pl.when(s < _NS - 1)
  def _():
    pltpu.sync_copy(zeros_hbm.at[pl.ds(s * _RPT, _RPT)],
                    acc.at[pl.ds(s * _RPT, _RPT)])
  @pl.when(s == _NS - 1)
  def _():
    pltpu.sync_copy(zeros_hbm.at[pl.ds(15 * _RPT, _RPT_LAST)],
                    acc.at[pl.ds(15 * _RPT, _RPT_LAST)])


def _copy_out(acc, out_hbm, c, s):
  @pl.when(s < _NS - 1)
  def _():
    pltpu.sync_copy(acc.at[pl.ds(s * _RPT, _RPT)],
                    out_hbm.at[c, pl.ds(s * _RPT, _RPT)])
  @pl.when(s == _NS - 1)
  def _():
    pltpu.sync_copy(acc.at[pl.ds(15 * _RPT, _RPT_LAST)],
                    out_hbm.at[c, pl.ds(15 * _RPT, _RPT_LAST)])


@functools.lru_cache(maxsize=None)
def _make_sc_scatter(D):
  mesh = plsc.VectorSubcoreMesh(core_axis_name="c", subcore_axis_name="s",
                                num_cores=_NC, num_subcores=_NS)

  @functools.partial(
      pl.kernel,
      mesh=mesh,
      compiler_params=pltpu.CompilerParams(use_tc_tiling_on_sc=False),
      out_type=jax.ShapeDtypeStruct((_NC, _N, D), jnp.float32),
      scratch_types=[
          pltpu.VMEM((_MAXCH, _CH), jnp.int32),
          pltpu.VMEM((_MAXCH, _CH), jnp.int32),
          [pltpu.VMEM((_CH, D), jnp.float32) for _ in range(_NBUF)],
          pltpu.VMEM_SHARED((_N, D), jnp.float32),
          [pltpu.SemaphoreType.DMA for _ in range(_NBUF)],
      ],
  )
  def scat(xs_hbm, src_hbm, dst_hbm, zeros_hbm, out_hbm,
           sidx, didx, rows, acc, gsem):
    c = lax.axis_index("c")
    s = lax.axis_index("s")
    wid = c * _NS + s
    _zero_acc(zeros_hbm, acc, s)
    pltpu.sync_copy(src_hbm.at[pl.ds(wid * _CPT, _CPT)],
                    sidx.at[pl.ds(0, _CPT)])
    pltpu.sync_copy(dst_hbm.at[pl.ds(wid * _CPT, _CPT)],
                    didx.at[pl.ds(0, _CPT)])
    @pl.when(wid < _EXTRA)
    def _():
      pltpu.sync_copy(src_hbm.at[pl.ds(_NT * _CPT + wid, 1)],
                      sidx.at[pl.ds(_CPT, 1)])
      pltpu.sync_copy(dst_hbm.at[pl.ds(_NT * _CPT + wid, 1)],
                      didx.at[pl.ds(_CPT, 1)])
    nch = jnp.where(wid < _EXTRA, _CPT + 1, _CPT)
    plsc.subcore_barrier()

    for b in range(_NBUF):
      pltpu.async_copy(xs_hbm.at[sidx.at[b]], rows[b], gsem[b])

    def group(g, carry):
      for b in range(_NBUF):
        j = g * _NBUF + b
        @pl.when(j < nch)
        def _():
          pltpu.make_async_copy(xs_hbm.at[sidx.at[j]], rows[b], gsem[b]).wait()
          pltpu.sync_copy(rows[b], acc.at[didx.at[j]], add=True)
          @pl.when(j + _NBUF < nch)
          def _():
            pltpu.async_copy(xs_hbm.at[sidx.at[j + _NBUF]], rows[b], gsem[b])
      return carry

    lax.fori_loop(0, (_MAXCH + _NBUF - 1) // _NBUF, group, 0, unroll=False)
    plsc.subcore_barrier()
    _copy_out(acc, out_hbm, c, s)

  return scat


@functools.lru_cache(maxsize=None)
def _make_sc_count(D):
  mesh = plsc.VectorSubcoreMesh(core_axis_name="c", subcore_axis_name="s",
                                num_cores=_NC, num_subcores=_NS)

  @functools.partial(
      pl.kernel,
      mesh=mesh,
      compiler_params=pltpu.CompilerParams(use_tc_tiling_on_sc=False),
      out_type=jax.ShapeDtypeStruct((_NC, _N, D), jnp.float32),
      scratch_types=[
          pltpu.VMEM((_MAXCH, _CH), jnp.int32),
          pltpu.VMEM((_CH, D), jnp.float32),
          pltpu.VMEM_SHARED((_N, D), jnp.float32),
          [pltpu.SemaphoreType.DMA for _ in range(_NBUF)],
      ],
  )
  def count(ones_hbm, dst_hbm, zeros_hbm, out_hbm, didx, ones_v, acc, ssem):
    c = lax.axis_index("c")
    s = lax.axis_index("s")
    wid = c * _NS + s
    _zero_acc(zeros_hbm, acc, s)
    pltpu.sync_copy(dst_hbm.at[pl.ds(wid * _CPT, _CPT)],
                    didx.at[pl.ds(0, _CPT)])
    @pl.when(wid < _EXTRA)
    def _():
      pltpu.sync_copy(dst_hbm.at[pl.ds(_NT * _CPT + wid, 1)],
                      didx.at[pl.ds(_CPT, 1)])
    nch = jnp.where(wid < _EXTRA, _CPT + 1, _CPT)
    pltpu.sync_copy(ones_hbm, ones_v)
    plsc.subcore_barrier()

    def group(g, carry):
      for b in range(_NBUF):
        j = g * _NBUF + b
        @pl.when(j < nch)
        def _():
          @pl.when(j >= _NBUF)
          def _():
            pltpu.make_async_copy(ones_v, acc.at[didx.at[j]], ssem[b]).wait()
          pltpu.async_copy(ones_v, acc.at[didx.at[j]], ssem[b], add=True)
      return carry

    lax.fori_loop(0, (_MAXCH + _NBUF - 1) // _NBUF, group, 0, unroll=False)
    for b in range(_NBUF):
      pltpu.make_async_copy(ones_v, acc.at[didx.at[0]], ssem[b]).wait()
    plsc.subcore_barrier()
    _copy_out(acc, out_hbm, c, s)

  return count


def _k1a_body(x_ref, w_ref, xw_ref):
  xw_ref[...] = jnp.dot(x_ref[...], w_ref[...],
                        preferred_element_type=jnp.float32)


def _k1b_body(xw_ref, d0_ref, d1_ref, xs_ref, dinv_ref):
  deg = d0_ref[...][:, 0:1] + d1_ref[...][:, 0:1] + 1.0
  dinv = lax.rsqrt(deg)
  xs_ref[...] = xw_ref[...] * dinv
  dinv_ref[...] = dinv


def _norm_affine(hc, bng, lng):
  m1 = jnp.mean(hc, axis=0, keepdims=True)
  v1 = jnp.maximum(jnp.mean(hc * hc, axis=0, keepdims=True) - m1 * m1, 0.0)
  a1 = bng * lax.rsqrt(v1 + _EPS)
  A = a1 * lax.rsqrt(a1 * a1 * v1 + _EPS)
  gv = jnp.mean(A * A * v1)
  F = A * lng * lax.rsqrt(gv + _EPS)
  return F, m1


def _klayer_body(p0_ref, p1_ref, xs_ref, dinv_ref, b_ref, bng_ref,
                 lng_ref, lnb_ref, w_ref, out_ref):
  dinv = dinv_ref[...]
  hc = (p0_ref[...] + p1_ref[...] + xs_ref[...]) * dinv + b_ref[...]
  F, m1 = _norm_affine(hc, bng_ref[...], lng_ref[...])
  h = jnp.maximum(F * (hc - m1) + lnb_ref[...], 0.0)
  xw = jnp.dot(h, w_ref[...], preferred_element_type=jnp.float32) * dinv
  if out_ref.shape[1] != xw.shape[1]:
    xw = jnp.broadcast_to(xw, out_ref.shape)
  out_ref[...] = xw


def _kfinal_body(p0_ref, p1_ref, xs_ref, dinv_ref, b_ref, bng_ref,
                 lng_ref, lnb_ref, out_ref):
  mp = p0_ref[...][:, 0:1] + p1_ref[...][:, 0:1] + xs_ref[...][:, 0:1]
  hc = mp * dinv_ref[...] + b_ref[...]
  F, m1 = _norm_affine(hc, bng_ref[...], lng_ref[...])
  out_ref[...] = F * (hc - m1) + lnb_ref[...]


def _tc_call(body, out_shape):
  return pl.pallas_call(body, out_shape=out_shape)


def kernel(x, edge_index, W1, b1, W2, b2, W3, b3, bn1_g, bn1_b, bn2_g,
           bn2_b, bn3_g, bn3_b, ln1_g, ln1_b, ln2_g, ln2_b, ln3_g, ln3_b):
  f32 = jnp.float32
  src = edge_index[0].reshape(_ROWS, _CH)
  dst = edge_index[1].reshape(_ROWS, _CH)
  zeros64 = jnp.zeros((_N, 64), f32)
  zeros8 = jnp.zeros((_N, 8), f32)
  ones8 = jnp.ones((_CH, 8), f32)

  sc64 = _make_sc_scatter(64)
  sc8 = _make_sc_scatter(8)

  degp = _make_sc_count(8)(ones8, dst, zeros8)
  xw1 = _tc_call(_k1a_body, jax.ShapeDtypeStruct((_N, 64), f32))(x, W1)

  xs1, dinv = _tc_call(
      _k1b_body,
      (jax.ShapeDtypeStruct((_N, 64), f32), jax.ShapeDtypeStruct((_N, 1), f32)),
  )(xw1, degp[0], degp[1])

  p1 = sc64(xs1, src, dst, zeros64)
  xs2 = _tc_call(_klayer_body, jax.ShapeDtypeStruct((_N, 64), f32))(
      p1[0], p1[1], xs1, dinv, b1.reshape(1, 64), bn1_g.reshape(1, 64),
      ln1_g.reshape(1, 64), ln1_b.reshape(1, 64), W2)

  p2 = sc64(xs2, src, dst, zeros64)
  xs3 = _tc_call(_klayer_body, jax.ShapeDtypeStruct((_N, 8), f32))(
      p2[0], p2[1], xs2, dinv, b2.reshape(1, 64), bn2_g.reshape(1, 64),
      ln2_g.reshape(1, 64), ln2_b.reshape(1, 64), W3)

  p3 = sc8(xs3, src, dst, zeros8)
  out = _tc_call(_kfinal_body, jax.ShapeDtypeStruct((_N, 1), f32))(
      p3[0], p3[1], xs3, dinv, b3.reshape(1, 1), bn3_g.reshape(1, 1),
      ln3_g.reshape(1, 1), ln3_b.reshape(1, 1))
  return out.reshape(-1)


# speedup vs baseline: 1.0916x; 1.0431x over previous
import functools

import jax
import jax.numpy as jnp
from jax import lax
from jax.experimental import pallas as pl
from jax.experimental.pallas import tpu as pltpu
from jax.experimental.pallas import tpu_sc as plsc

_N = 10000
_E = 320000
_EPS = 1e-5

_NC = 2
_NS = 16
_NT = _NC * _NS
_CH = 128
_ROWS = _E // _CH
_CPT = _ROWS // _NT
_EXTRA = _ROWS - _NT * _CPT
_MAXCH = _CPT + 1
_RPT = 624
_RPT_LAST = _N - 15 * _RPT

_NBUF = 4


def _zero_acc(zeros_hbm, acc, s):
  @pl.when(s < _NS - 1)
  def _():
    pltpu.sync_copy(zeros_hbm.at[pl.ds(s * _RPT, _RPT)],
                    acc.at[pl.ds(s * _RPT, _RPT)])
  @pl.when(s == _NS - 1)
  def _():
    pltpu.sync_copy(zeros_hbm.at[pl.ds(15 * _RPT, _RPT_LAST)],
                    acc.at[pl.ds(15 * _RPT, _RPT_LAST)])


def _copy_out(acc, out_hbm, c, s):
  @pl.when(s < _NS - 1)
  def _():
    pltpu.sync_copy(acc.at[pl.ds(s * _RPT, _RPT)],
                    out_hbm.at[c, pl.ds(s * _RPT, _RPT)])
  @pl.when(s == _NS - 1)
  def _():
    pltpu.sync_copy(acc.at[pl.ds(15 * _RPT, _RPT_LAST)],
                    out_hbm.at[c, pl.ds(15 * _RPT, _RPT_LAST)])


@functools.lru_cache(maxsize=None)
def _make_sc_scatter(D):
  mesh = plsc.VectorSubcoreMesh(core_axis_name="c", subcore_axis_name="s",
                                num_cores=_NC, num_subcores=_NS)

  @functools.partial(
      pl.kernel,
      mesh=mesh,
      compiler_params=pltpu.CompilerParams(use_tc_tiling_on_sc=False),
      out_type=jax.ShapeDtypeStruct((_NC, _N, D), jnp.float32),
      scratch_types=[
          pltpu.VMEM((_MAXCH, _CH), jnp.int32),
          pltpu.VMEM((_MAXCH, _CH), jnp.int32),
          [pltpu.VMEM((_CH, D), jnp.float32) for _ in range(_NBUF)],
          pltpu.VMEM_SHARED((_N, D), jnp.float32),
          [pltpu.SemaphoreType.DMA for _ in range(_NBUF)],
      ],
  )
  def scat(xs_hbm, src_hbm, dst_hbm, zeros_hbm, out_hbm,
           sidx, didx, rows, acc, gsem):
    c = lax.axis_index("c")
    s = lax.axis_index("s")
    wid = c * _NS + s
    _zero_acc(zeros_hbm, acc, s)
    pltpu.sync_copy(src_hbm.at[pl.ds(wid * _CPT, _CPT)],
                    sidx.at[pl.ds(0, _CPT)])
    pltpu.sync_copy(dst_hbm.at[pl.ds(wid * _CPT, _CPT)],
                    didx.at[pl.ds(0, _CPT)])
    @pl.when(wid < _EXTRA)
    def _():
      pltpu.sync_copy(src_hbm.at[pl.ds(_NT * _CPT + wid, 1)],
                      sidx.at[pl.ds(_CPT, 1)])
      pltpu.sync_copy(dst_hbm.at[pl.ds(_NT * _CPT + wid, 1)],
                      didx.at[pl.ds(_CPT, 1)])
    nch = jnp.where(wid < _EXTRA, _CPT + 1, _CPT)
    plsc.subcore_barrier()

    for b in range(_NBUF):
      pltpu.async_copy(xs_hbm.at[sidx.at[b]], rows[b], gsem[b])

    def group(g, carry):
      for b in range(_NBUF):
        j = g * _NBUF + b
        @pl.when(j < nch)
        def _():
          pltpu.make_async_copy(xs_hbm.at[sidx.at[j]], rows[b], gsem[b]).wait()
          pltpu.sync_copy(rows[b], acc.at[didx.at[j]], add=True)
          @pl.when(j + _NBUF < nch)
          def _():
            pltpu.async_copy(xs_hbm.at[sidx.at[j + _NBUF]], rows[b], gsem[b])
      return carry

    lax.fori_loop(0, (_MAXCH + _NBUF - 1) // _NBUF, group, 0, unroll=False)
    plsc.subcore_barrier()
    _copy_out(acc, out_hbm, c, s)

  return scat


@functools.lru_cache(maxsize=None)
def _make_sc_count(D):
  mesh = plsc.VectorSubcoreMesh(core_axis_name="c", subcore_axis_name="s",
                                num_cores=_NC, num_subcores=_NS)

  @functools.partial(
      pl.kernel,
      mesh=mesh,
      compiler_params=pltpu.CompilerParams(use_tc_tiling_on_sc=False),
      out_type=jax.ShapeDtypeStruct((_NC, _N, D), jnp.float32),
      scratch_types=[
          pltpu.VMEM((_MAXCH, _CH), jnp.int32),
          pltpu.VMEM((_CH, D), jnp.float32),
          pltpu.VMEM_SHARED((_N, D), jnp.float32),
          [pltpu.SemaphoreType.DMA for _ in range(_NBUF)],
      ],
  )
  def count(ones_hbm, dst_hbm, zeros_hbm, out_hbm, didx, ones_v, acc, ssem):
    c = lax.axis_index("c")
    s = lax.axis_index("s")
    wid = c * _NS + s
    _zero_acc(zeros_hbm, acc, s)
    pltpu.sync_copy(dst_hbm.at[pl.ds(wid * _CPT, _CPT)],
                    didx.at[pl.ds(0, _CPT)])
    @pl.when(wid < _EXTRA)
    def _():
      pltpu.sync_copy(dst_hbm.at[pl.ds(_NT * _CPT + wid, 1)],
                      didx.at[pl.ds(_CPT, 1)])
    nch = jnp.where(wid < _EXTRA, _CPT + 1, _CPT)
    pltpu.sync_copy(ones_hbm, ones_v)
    plsc.subcore_barrier()

    def group(g, carry):
      for b in range(_NBUF):
        j = g * _NBUF + b
        @pl.when(j < nch)
        def _():
          @pl.when(j >= _NBUF)
          def _():
            pltpu.make_async_copy(ones_v, acc.at[didx.at[j]], ssem[b]).wait()
          pltpu.async_copy(ones_v, acc.at[didx.at[j]], ssem[b], add=True)
      return carry

    lax.fori_loop(0, (_MAXCH + _NBUF - 1) // _NBUF, group, 0, unroll=False)
    for b in range(_NBUF):
      pltpu.make_async_copy(ones_v, acc.at[didx.at[0]], ssem[b]).wait()
    plsc.subcore_barrier()
    _copy_out(acc, out_hbm, c, s)

  return count


def _k1a_body(x_ref, w_ref, xw_ref):
  xw_ref[...] = jnp.dot(x_ref[...], w_ref[...],
                        preferred_element_type=jnp.float32)


def _k1b_body(xw_ref, d0_ref, d1_ref, xs_ref, dinv_ref):
  deg = d0_ref[...][:, 0:1] + d1_ref[...][:, 0:1] + 1.0
  dinv = lax.rsqrt(deg)
  xs_ref[...] = xw_ref[...] * dinv
  dinv_ref[...] = dinv


def _norm_affine(hc, bng, lng):
  m1 = jnp.mean(hc, axis=0, keepdims=True)
  v1 = jnp.maximum(jnp.mean(hc * hc, axis=0, keepdims=True) - m1 * m1, 0.0)
  a1 = bng * lax.rsqrt(v1 + _EPS)
  A = a1 * lax.rsqrt(a1 * a1 * v1 + _EPS)
  gv = jnp.mean(A * A * v1)
  F = A * lng * lax.rsqrt(gv + _EPS)
  return F, m1


def _klayer_body(p0_ref, p1_ref, xs_ref, dinv_ref, b_ref, bng_ref,
                 lng_ref, lnb_ref, w_ref, out_ref):
  dinv = dinv_ref[...]
  hc = (p0_ref[...] + p1_ref[...] + xs_ref[...]) * dinv + b_ref[...]
  F, m1 = _norm_affine(hc, bng_ref[...], lng_ref[...])
  h = jnp.maximum(F * (hc - m1) + lnb_ref[...], 0.0)
  xw = jnp.dot(h, w_ref[...], preferred_element_type=jnp.float32) * dinv
  if out_ref.shape[1] != xw.shape[1]:
    xw = jnp.broadcast_to(xw, out_ref.shape)
  out_ref[...] = xw


def _kfinal_body(p0_ref, p1_ref, xs_ref, dinv_ref, b_ref, bng_ref,
                 lng_ref, lnb_ref, out_ref):
  hc = (p0_ref[...] + p1_ref[...] + xs_ref[...]) * dinv_ref[...] + b_ref[0, 0]
  m1 = jnp.mean(hc)
  v1 = jnp.maximum(jnp.mean(hc * hc) - m1 * m1, 0.0)
  a1 = bng_ref[0, 0] * lax.rsqrt(v1 + _EPS)
  A = a1 * lax.rsqrt(a1 * a1 * v1 + _EPS)
  F = A * lng_ref[0, 0] * lax.rsqrt(A * A * v1 + _EPS)
  out_ref[...] = F * (hc - m1) + lnb_ref[0, 0]


def _tc_call(body, out_shape):
  return pl.pallas_call(body, out_shape=out_shape)


def kernel(x, edge_index, W1, b1, W2, b2, W3, b3, bn1_g, bn1_b, bn2_g,
           bn2_b, bn3_g, bn3_b, ln1_g, ln1_b, ln2_g, ln2_b, ln3_g, ln3_b):
  f32 = jnp.float32
  src = edge_index[0].reshape(_ROWS, _CH)
  dst = edge_index[1].reshape(_ROWS, _CH)
  zeros64 = jnp.zeros((_N, 64), f32)
  zeros8 = jnp.zeros((_N, 8), f32)
  ones8 = jnp.ones((_CH, 8), f32)

  sc64 = _make_sc_scatter(64)
  sc8 = _make_sc_scatter(8)

  degp = _make_sc_count(8)(ones8, dst, zeros8)
  xw1 = _tc_call(_k1a_body, jax.ShapeDtypeStruct((_N, 64), f32))(x, W1)

  xs1, dinv = _tc_call(
      _k1b_body,
      (jax.ShapeDtypeStruct((_N, 64), f32), jax.ShapeDtypeStruct((_N, 1), f32)),
  )(xw1, degp[0], degp[1])

  p1 = sc64(xs1, src, dst, zeros64)
  xs2 = _tc_call(_klayer_body, jax.ShapeDtypeStruct((_N, 64), f32))(
      p1[0], p1[1], xs1, dinv, b1.reshape(1, 64), bn1_g.reshape(1, 64),
      ln1_g.reshape(1, 64), ln1_b.reshape(1, 64), W2)

  p2 = sc64(xs2, src, dst, zeros64)
  xs3 = _tc_call(_klayer_body, jax.ShapeDtypeStruct((_N, 8), f32))(
      p2[0], p2[1], xs2, dinv, b2.reshape(1, 64), bn2_g.reshape(1, 64),
      ln2_g.reshape(1, 64), ln2_b.reshape(1, 64), W3)

  p3 = sc8(xs3, src, dst, zeros8)
  out = _tc_call(_kfinal_body, jax.ShapeDtypeStruct((80, 125), f32))(
      p3[0, :, 0].reshape(80, 125), p3[1, :, 0].reshape(80, 125),
      xs3[:, 0].reshape(80, 125), dinv.reshape(80, 125),
      b3.reshape(1, 1), bn3_g.reshape(1, 1),
      ln3_g.reshape(1, 1), ln3_b.reshape(1, 1))
  return out.reshape(-1)


# speedup vs baseline: 1.1112x; 1.0180x over previous
import functools

import jax
import jax.numpy as jnp
from jax import lax
from jax.experimental import pallas as pl
from jax.experimental.pallas import tpu as pltpu
from jax.experimental.pallas import tpu_sc as plsc

_N = 10000
_E = 320000
_EPS = 1e-5

_NC = 2
_NS = 16
_NT = _NC * _NS
_CH = 128
_ROWS = _E // _CH
_CPT = _ROWS // _NT
_EXTRA = _ROWS - _NT * _CPT
_MAXCH = _CPT + 1
_RPT = 624
_RPT_LAST = _N - 15 * _RPT

_NBUF = 6


def _zero_acc(zeros_hbm, acc, s):
  @pl.when(s < _NS - 1)
  def _():
    pltpu.sync_copy(zeros_hbm.at[pl.ds(s * _RPT, _RPT)],
                    acc.at[pl.ds(s * _RPT, _RPT)])
  @pl.when(s == _NS - 1)
  def _():
    pltpu.sync_copy(zeros_hbm.at[pl.ds(15 * _RPT, _RPT_LAST)],
                    acc.at[pl.ds(15 * _RPT, _RPT_LAST)])


def _copy_out(acc, out_hbm, c, s):
  @pl.when(s < _NS - 1)
  def _():
    pltpu.sync_copy(acc.at[pl.ds(s * _RPT, _RPT)],
                    out_hbm.at[c, pl.ds(s * _RPT, _RPT)])
  @pl.when(s == _NS - 1)
  def _():
    pltpu.sync_copy(acc.at[pl.ds(15 * _RPT, _RPT_LAST)],
                    out_hbm.at[c, pl.ds(15 * _RPT, _RPT_LAST)])


@functools.lru_cache(maxsize=None)
def _make_sc_scatter(D):
  mesh = plsc.VectorSubcoreMesh(core_axis_name="c", subcore_axis_name="s",
                                num_cores=_NC, num_subcores=_NS)

  @functools.partial(
      pl.kernel,
      mesh=mesh,
      compiler_params=pltpu.CompilerParams(use_tc_tiling_on_sc=False),
      out_type=jax.ShapeDtypeStruct((_NC, _N, D), jnp.float32),
      scratch_types=[
          pltpu.VMEM((_MAXCH, _CH), jnp.int32),
          pltpu.VMEM((_MAXCH, _CH), jnp.int32),
          [pltpu.VMEM((_CH, D), jnp.float32) for _ in range(_NBUF)],
          pltpu.VMEM_SHARED((_N, D), jnp.float32),
          [pltpu.SemaphoreType.DMA for _ in range(_NBUF)],
      ],
  )
  def scat(xs_hbm, src_hbm, dst_hbm, zeros_hbm, out_hbm,
           sidx, didx, rows, acc, gsem):
    c = lax.axis_index("c")
    s = lax.axis_index("s")
    wid = c * _NS + s
    _zero_acc(zeros_hbm, acc, s)
    pltpu.sync_copy(src_hbm.at[pl.ds(wid * _CPT, _CPT)],
                    sidx.at[pl.ds(0, _CPT)])
    pltpu.sync_copy(dst_hbm.at[pl.ds(wid * _CPT, _CPT)],
                    didx.at[pl.ds(0, _CPT)])
    @pl.when(wid < _EXTRA)
    def _():
      pltpu.sync_copy(src_hbm.at[pl.ds(_NT * _CPT + wid, 1)],
                      sidx.at[pl.ds(_CPT, 1)])
      pltpu.sync_copy(dst_hbm.at[pl.ds(_NT * _CPT + wid, 1)],
                      didx.at[pl.ds(_CPT, 1)])
    nch = jnp.where(wid < _EXTRA, _CPT + 1, _CPT)
    plsc.subcore_barrier()

    for b in range(_NBUF):
      pltpu.async_copy(xs_hbm.at[sidx.at[b]], rows[b], gsem[b])

    def group(g, carry):
      for b in range(_NBUF):
        j = g * _NBUF + b
        @pl.when(j < nch)
        def _():
          pltpu.make_async_copy(xs_hbm.at[sidx.at[j]], rows[b], gsem[b]).wait()
          pltpu.sync_copy(rows[b], acc.at[didx.at[j]], add=True)
          @pl.when(j + _NBUF < nch)
          def _():
            pltpu.async_copy(xs_hbm.at[sidx.at[j + _NBUF]], rows[b], gsem[b])
      return carry

    lax.fori_loop(0, (_MAXCH + _NBUF - 1) // _NBUF, group, 0, unroll=False)
    plsc.subcore_barrier()
    _copy_out(acc, out_hbm, c, s)

  return scat


@functools.lru_cache(maxsize=None)
def _make_sc_count(D):
  mesh = plsc.VectorSubcoreMesh(core_axis_name="c", subcore_axis_name="s",
                                num_cores=_NC, num_subcores=_NS)

  @functools.partial(
      pl.kernel,
      mesh=mesh,
      compiler_params=pltpu.CompilerParams(use_tc_tiling_on_sc=False),
      out_type=jax.ShapeDtypeStruct((_NC, _N, D), jnp.float32),
      scratch_types=[
          pltpu.VMEM((_MAXCH, _CH), jnp.int32),
          pltpu.VMEM((_CH, D), jnp.float32),
          pltpu.VMEM_SHARED((_N, D), jnp.float32),
          [pltpu.SemaphoreType.DMA for _ in range(_NBUF)],
      ],
  )
  def count(ones_hbm, dst_hbm, zeros_hbm, out_hbm, didx, ones_v, acc, ssem):
    c = lax.axis_index("c")
    s = lax.axis_index("s")
    wid = c * _NS + s
    _zero_acc(zeros_hbm, acc, s)
    pltpu.sync_copy(dst_hbm.at[pl.ds(wid * _CPT, _CPT)],
                    didx.at[pl.ds(0, _CPT)])
    @pl.when(wid < _EXTRA)
    def _():
      pltpu.sync_copy(dst_hbm.at[pl.ds(_NT * _CPT + wid, 1)],
                      didx.at[pl.ds(_CPT, 1)])
    nch = jnp.where(wid < _EXTRA, _CPT + 1, _CPT)
    pltpu.sync_copy(ones_hbm, ones_v)
    plsc.subcore_barrier()

    def group(g, carry):
      for b in range(_NBUF):
        j = g * _NBUF + b
        @pl.when(j < nch)
        def _():
          @pl.when(j >= _NBUF)
          def _():
            pltpu.make_async_copy(ones_v, acc.at[didx.at[j]], ssem[b]).wait()
          pltpu.async_copy(ones_v, acc.at[didx.at[j]], ssem[b], add=True)
      return carry

    lax.fori_loop(0, (_MAXCH + _NBUF - 1) // _NBUF, group, 0, unroll=False)
    for b in range(_NBUF):
      pltpu.make_async_copy(ones_v, acc.at[didx.at[0]], ssem[b]).wait()
    plsc.subcore_barrier()
    _copy_out(acc, out_hbm, c, s)

  return count


def _k1a_body(x_ref, w_ref, xw_ref):
  xw_ref[...] = jnp.dot(x_ref[...], w_ref[...],
                        preferred_element_type=jnp.float32)


def _k1b_body(xw_ref, d0_ref, d1_ref, xs_ref, dinv_ref):
  deg = d0_ref[...][:, 0:1] + d1_ref[...][:, 0:1] + 1.0
  dinv = lax.rsqrt(deg)
  xs_ref[...] = xw_ref[...] * dinv
  dinv_ref[...] = dinv


def _norm_affine(hc, bng, lng):
  m1 = jnp.mean(hc, axis=0, keepdims=True)
  v1 = jnp.maximum(jnp.mean(hc * hc, axis=0, keepdims=True) - m1 * m1, 0.0)
  a1 = bng * lax.rsqrt(v1 + _EPS)
  A = a1 * lax.rsqrt(a1 * a1 * v1 + _EPS)
  gv = jnp.mean(A * A * v1)
  F = A * lng * lax.rsqrt(gv + _EPS)
  return F, m1


def _klayer_body(p0_ref, p1_ref, xs_ref, dinv_ref, b_ref, bng_ref,
                 lng_ref, lnb_ref, w_ref, out_ref):
  dinv = dinv_ref[...]
  hc = (p0_ref[...] + p1_ref[...] + xs_ref[...]) * dinv + b_ref[...]
  F, m1 = _norm_affine(hc, bng_ref[...], lng_ref[...])
  h = jnp.maximum(F * (hc - m1) + lnb_ref[...], 0.0)
  xw = jnp.dot(h, w_ref[...], preferred_element_type=jnp.float32) * dinv
  if out_ref.shape[1] != xw.shape[1]:
    xw = jnp.broadcast_to(xw, out_ref.shape)
  out_ref[...] = xw


def _kfinal_body(p0_ref, p1_ref, xs_ref, dinv_ref, b_ref, bng_ref,
                 lng_ref, lnb_ref, out_ref):
  hc = (p0_ref[...] + p1_ref[...] + xs_ref[...]) * dinv_ref[...] + b_ref[0, 0]
  m1 = jnp.mean(hc)
  v1 = jnp.maximum(jnp.mean(hc * hc) - m1 * m1, 0.0)
  a1 = bng_ref[0, 0] * lax.rsqrt(v1 + _EPS)
  A = a1 * lax.rsqrt(a1 * a1 * v1 + _EPS)
  F = A * lng_ref[0, 0] * lax.rsqrt(A * A * v1 + _EPS)
  out_ref[...] = F * (hc - m1) + lnb_ref[0, 0]


def _tc_call(body, out_shape):
  return pl.pallas_call(body, out_shape=out_shape)


def kernel(x, edge_index, W1, b1, W2, b2, W3, b3, bn1_g, bn1_b, bn2_g,
           bn2_b, bn3_g, bn3_b, ln1_g, ln1_b, ln2_g, ln2_b, ln3_g, ln3_b):
  f32 = jnp.float32
  src = edge_index[0].reshape(_ROWS, _CH)
  dst = edge_index[1].reshape(_ROWS, _CH)
  zeros64 = jnp.zeros((_N, 64), f32)
  zeros8 = jnp.zeros((_N, 8), f32)
  ones8 = jnp.ones((_CH, 8), f32)

  sc64 = _make_sc_scatter(64)
  sc8 = _make_sc_scatter(8)

  degp = _make_sc_count(8)(ones8, dst, zeros8)
  xw1 = _tc_call(_k1a_body, jax.ShapeDtypeStruct((_N, 64), f32))(x, W1)

  xs1, dinv = _tc_call(
      _k1b_body,
      (jax.ShapeDtypeStruct((_N, 64), f32), jax.ShapeDtypeStruct((_N, 1), f32)),
  )(xw1, degp[0], degp[1])

  p1 = sc64(xs1, src, dst, zeros64)
  xs2 = _tc_call(_klayer_body, jax.ShapeDtypeStruct((_N, 64), f32))(
      p1[0], p1[1], xs1, dinv, b1.reshape(1, 64), bn1_g.reshape(1, 64),
      ln1_g.reshape(1, 64), ln1_b.reshape(1, 64), W2)

  p2 = sc64(xs2, src, dst, zeros64)
  xs3 = _tc_call(_klayer_body, jax.ShapeDtypeStruct((_N, 8), f32))(
      p2[0], p2[1], xs2, dinv, b2.reshape(1, 64), bn2_g.reshape(1, 64),
      ln2_g.reshape(1, 64), ln2_b.reshape(1, 64), W3)

  p3 = sc8(xs3, src, dst, zeros8)
  out = _tc_call(_kfinal_body, jax.ShapeDtypeStruct((80, 125), f32))(
      p3[0, :, 0].reshape(80, 125), p3[1, :, 0].reshape(80, 125),
      xs3[:, 0].reshape(80, 125), dinv.reshape(80, 125),
      b3.reshape(1, 1), bn3_g.reshape(1, 1),
      ln3_g.reshape(1, 1), ln3_b.reshape(1, 1))
  return out.reshape(-1)


# speedup vs baseline: 1.1127x; 1.0014x over previous
import functools

import jax
import jax.numpy as jnp
from jax import lax
from jax.experimental import pallas as pl
from jax.experimental.pallas import tpu as pltpu
from jax.experimental.pallas import tpu_sc as plsc

_N = 10000
_E = 320000
_EPS = 1e-5

_NC = 2
_NS = 16
_NT = _NC * _NS
_CH = 128
_ROWS = _E // _CH
_CPT = _ROWS // _NT
_EXTRA = _ROWS - _NT * _CPT
_MAXCH = _CPT + 1
_RPT = 624
_RPT_LAST = _N - 15 * _RPT

_NBUF = 8


def _zero_acc(zeros_hbm, acc, s):
  @pl.when(s < _NS - 1)
  def _():
    pltpu.sync_copy(zeros_hbm.at[pl.ds(s * _RPT, _RPT)],
                    acc.at[pl.ds(s * _RPT, _RPT)])
  @pl.when(s == _NS - 1)
  def _():
    pltpu.sync_copy(zeros_hbm.at[pl.ds(15 * _RPT, _RPT_LAST)],
                    acc.at[pl.ds(15 * _RPT, _RPT_LAST)])


def _copy_out(acc, out_hbm, c, s):
  @pl.when(s < _NS - 1)
  def _():
    pltpu.sync_copy(acc.at[pl.ds(s * _RPT, _RPT)],
                    out_hbm.at[c, pl.ds(s * _RPT, _RPT)])
  @pl.when(s == _NS - 1)
  def _():
    pltpu.sync_copy(acc.at[pl.ds(15 * _RPT, _RPT_LAST)],
                    out_hbm.at[c, pl.ds(15 * _RPT, _RPT_LAST)])


@functools.lru_cache(maxsize=None)
def _make_sc_scatter(D):
  mesh = plsc.VectorSubcoreMesh(core_axis_name="c", subcore_axis_name="s",
                                num_cores=_NC, num_subcores=_NS)

  @functools.partial(
      pl.kernel,
      mesh=mesh,
      compiler_params=pltpu.CompilerParams(use_tc_tiling_on_sc=False),
      out_type=jax.ShapeDtypeStruct((_NC, _N, D), jnp.float32),
      scratch_types=[
          pltpu.VMEM((_MAXCH, _CH), jnp.int32),
          pltpu.VMEM((_MAXCH, _CH), jnp.int32),
          [pltpu.VMEM((_CH, D), jnp.float32) for _ in range(_NBUF)],
          pltpu.VMEM_SHARED((_N, D), jnp.float32),
          [pltpu.SemaphoreType.DMA for _ in range(_NBUF)],
      ],
  )
  def scat(xs_hbm, src_hbm, dst_hbm, zeros_hbm, out_hbm,
           sidx, didx, rows, acc, gsem):
    c = lax.axis_index("c")
    s = lax.axis_index("s")
    wid = c * _NS + s
    _zero_acc(zeros_hbm, acc, s)
    pltpu.sync_copy(src_hbm.at[pl.ds(wid * _CPT, _CPT)],
                    sidx.at[pl.ds(0, _CPT)])
    pltpu.sync_copy(dst_hbm.at[pl.ds(wid * _CPT, _CPT)],
                    didx.at[pl.ds(0, _CPT)])
    @pl.when(wid < _EXTRA)
    def _():
      pltpu.sync_copy(src_hbm.at[pl.ds(_NT * _CPT + wid, 1)],
                      sidx.at[pl.ds(_CPT, 1)])
      pltpu.sync_copy(dst_hbm.at[pl.ds(_NT * _CPT + wid, 1)],
                      didx.at[pl.ds(_CPT, 1)])
    nch = jnp.where(wid < _EXTRA, _CPT + 1, _CPT)
    plsc.subcore_barrier()

    for b in range(_NBUF):
      pltpu.async_copy(xs_hbm.at[sidx.at[b]], rows[b], gsem[b])

    def group(g, carry):
      for b in range(_NBUF):
        j = g * _NBUF + b
        @pl.when(j < nch)
        def _():
          pltpu.make_async_copy(xs_hbm.at[sidx.at[j]], rows[b], gsem[b]).wait()
          pltpu.sync_copy(rows[b], acc.at[didx.at[j]], add=True)
          @pl.when(j + _NBUF < nch)
          def _():
            pltpu.async_copy(xs_hbm.at[sidx.at[j + _NBUF]], rows[b], gsem[b])
      return carry

    lax.fori_loop(0, (_MAXCH + _NBUF - 1) // _NBUF, group, 0, unroll=False)
    plsc.subcore_barrier()
    _copy_out(acc, out_hbm, c, s)

  return scat


@functools.lru_cache(maxsize=None)
def _make_sc_count(D):
  mesh = plsc.VectorSubcoreMesh(core_axis_name="c", subcore_axis_name="s",
                                num_cores=_NC, num_subcores=_NS)

  @functools.partial(
      pl.kernel,
      mesh=mesh,
      compiler_params=pltpu.CompilerParams(use_tc_tiling_on_sc=False),
      out_type=jax.ShapeDtypeStruct((_NC, _N, D), jnp.float32),
      scratch_types=[
          pltpu.VMEM((_MAXCH, _CH), jnp.int32),
          pltpu.VMEM((_CH, D), jnp.float32),
          pltpu.VMEM_SHARED((_N, D), jnp.float32),
          [pltpu.SemaphoreType.DMA for _ in range(_NBUF)],
      ],
  )
  def count(ones_hbm, dst_hbm, zeros_hbm, out_hbm, didx, ones_v, acc, ssem):
    c = lax.axis_index("c")
    s = lax.axis_index("s")
    wid = c * _NS + s
    _zero_acc(zeros_hbm, acc, s)
    pltpu.sync_copy(dst_hbm.at[pl.ds(wid * _CPT, _CPT)],
                    didx.at[pl.ds(0, _CPT)])
    @pl.when(wid < _EXTRA)
    def _():
      pltpu.sync_copy(dst_hbm.at[pl.ds(_NT * _CPT + wid, 1)],
                      didx.at[pl.ds(_CPT, 1)])
    nch = jnp.where(wid < _EXTRA, _CPT + 1, _CPT)
    pltpu.sync_copy(ones_hbm, ones_v)
    plsc.subcore_barrier()

    def group(g, carry):
      for b in range(_NBUF):
        j = g * _NBUF + b
        @pl.when(j < nch)
        def _():
          @pl.when(j >= _NBUF)
          def _():
            pltpu.make_async_copy(ones_v, acc.at[didx.at[j]], ssem[b]).wait()
          pltpu.async_copy(ones_v, acc.at[didx.at[j]], ssem[b], add=True)
      return carry

    lax.fori_loop(0, (_MAXCH + _NBUF - 1) // _NBUF, group, 0, unroll=False)
    for b in range(_NBUF):
      pltpu.make_async_copy(ones_v, acc.at[didx.at[0]], ssem[b]).wait()
    plsc.subcore_barrier()
    _copy_out(acc, out_hbm, c, s)

  return count


def _k1a_body(x_ref, w_ref, xw_ref):
  xw_ref[...] = jnp.dot(x_ref[...], w_ref[...],
                        preferred_element_type=jnp.float32)


def _k1b_body(xw_ref, d0_ref, d1_ref, xs_ref, dinv_ref):
  deg = d0_ref[...][:, 0:1] + d1_ref[...][:, 0:1] + 1.0
  dinv = lax.rsqrt(deg)
  xs_ref[...] = xw_ref[...] * dinv
  dinv_ref[...] = dinv


def _norm_affine(hc, bng, lng):
  m1 = jnp.mean(hc, axis=0, keepdims=True)
  v1 = jnp.maximum(jnp.mean(hc * hc, axis=0, keepdims=True) - m1 * m1, 0.0)
  a1 = bng * lax.rsqrt(v1 + _EPS)
  A = a1 * lax.rsqrt(a1 * a1 * v1 + _EPS)
  gv = jnp.mean(A * A * v1)
  F = A * lng * lax.rsqrt(gv + _EPS)
  return F, m1


def _klayer_body(p0_ref, p1_ref, xs_ref, dinv_ref, b_ref, bng_ref,
                 lng_ref, lnb_ref, w_ref, out_ref):
  dinv = dinv_ref[...]
  hc = (p0_ref[...] + p1_ref[...] + xs_ref[...]) * dinv + b_ref[...]
  F, m1 = _norm_affine(hc, bng_ref[...], lng_ref[...])
  h = jnp.maximum(F * (hc - m1) + lnb_ref[...], 0.0)
  xw = jnp.dot(h, w_ref[...], preferred_element_type=jnp.float32) * dinv
  if out_ref.shape[1] != xw.shape[1]:
    xw = jnp.broadcast_to(xw, out_ref.shape)
  out_ref[...] = xw


def _kfinal_body(p0_ref, p1_ref, xs_ref, dinv_ref, b_ref, bng_ref,
                 lng_ref, lnb_ref, out_ref):
  hc = (p0_ref[...] + p1_ref[...] + xs_ref[...]) * dinv_ref[...] + b_ref[0, 0]
  m1 = jnp.mean(hc)
  v1 = jnp.maximum(jnp.mean(hc * hc) - m1 * m1, 0.0)
  a1 = bng_ref[0, 0] * lax.rsqrt(v1 + _EPS)
  A = a1 * lax.rsqrt(a1 * a1 * v1 + _EPS)
  F = A * lng_ref[0, 0] * lax.rsqrt(A * A * v1 + _EPS)
  out_ref[...] = F * (hc - m1) + lnb_ref[0, 0]


def _tc_call(body, out_shape):
  return pl.pallas_call(body, out_shape=out_shape)


def kernel(x, edge_index, W1, b1, W2, b2, W3, b3, bn1_g, bn1_b, bn2_g,
           bn2_b, bn3_g, bn3_b, ln1_g, ln1_b, ln2_g, ln2_b, ln3_g, ln3_b):
  f32 = jnp.float32
  src = edge_index[0].reshape(_ROWS, _CH)
  dst = edge_index[1].reshape(_ROWS, _CH)
  zeros64 = jnp.zeros((_N, 64), f32)
  zeros8 = jnp.zeros((_N, 8), f32)
  ones8 = jnp.ones((_CH, 8), f32)

  sc64 = _make_sc_scatter(64)
  sc8 = _make_sc_scatter(8)

  degp = _make_sc_count(8)(ones8, dst, zeros8)
  xw1 = _tc_call(_k1a_body, jax.ShapeDtypeStruct((_N, 64), f32))(x, W1)

  xs1, dinv = _tc_call(
      _k1b_body,
      (jax.ShapeDtypeStruct((_N, 64), f32), jax.ShapeDtypeStruct((_N, 1), f32)),
  )(xw1, degp[0], degp[1])

  p1 = sc64(xs1, src, dst, zeros64)
  xs2 = _tc_call(_klayer_body, jax.ShapeDtypeStruct((_N, 64), f32))(
      p1[0], p1[1], xs1, dinv, b1.reshape(1, 64), bn1_g.reshape(1, 64),
      ln1_g.reshape(1, 64), ln1_b.reshape(1, 64), W2)

  p2 = sc64(xs2, src, dst, zeros64)
  xs3 = _tc_call(_klayer_body, jax.ShapeDtypeStruct((_N, 8), f32))(
      p2[0], p2[1], xs2, dinv, b2.reshape(1, 64), bn2_g.reshape(1, 64),
      ln2_g.reshape(1, 64), ln2_b.reshape(1, 64), W3)

  p3 = sc8(xs3, src, dst, zeros8)
  out = _tc_call(_kfinal_body, jax.ShapeDtypeStruct((80, 125), f32))(
      p3[0, :, 0].reshape(80, 125), p3[1, :, 0].reshape(80, 125),
      xs3[:, 0].reshape(80, 125), dinv.reshape(80, 125),
      b3.reshape(1, 1), bn3_g.reshape(1, 1),
      ln3_g.reshape(1, 1), ln3_b.reshape(1, 1))
  return out.reshape(-1)
